# Initial kernel scaffold; baseline (speedup 1.0000x reference)
import jax
import jax.numpy as jnp
from jax.experimental import pallas as pl


def kernel(x, edge_index, W1, att_src1, att_dst1, b1, W2, att_src2, att_dst2, b2):
    raise NotImplementedError("write your pallas kernel here")


# speedup vs baseline: 40.2878x; 40.2878x over previous
import functools
import jax
import jax.numpy as jnp
from jax import lax
from jax.experimental import pallas as pl
from jax.experimental.pallas import tpu as pltpu
from jax.experimental.pallas import tpu_sc as plsc

N = 10000
E = 320000
NFEAT = 128
NHID = 16
NCLASS = 40
HEADS = 8
H1 = HEADS * NHID
R1 = 144
R2 = 48

NC, NS, LN = 2, 16, 16
NW = NC * NS
EPT = E // NW
K = 80
NB = EPT // K
NPAD = 10240
RPT = NPAD // NS
ZR = 128

BLK = 2000
f32 = jnp.float32
i32 = jnp.int32


def _head_bcast_mask(rows, cols, transpose=False):
  a = lax.broadcasted_iota(i32, (rows, cols), 0)
  b = lax.broadcasted_iota(i32, (rows, cols), 1)
  if transpose:
    m = a == b // NHID
  else:
    m = a // NHID == b
  return m.astype(f32)


def _phase_a_body(x_ref, w1_ref, as1_ref, ad1_ref, row1_ref, adt_ref):
  h = jnp.dot(x_ref[...], w1_ref[...], preferred_element_type=f32)
  m = _head_bcast_mask(H1, HEADS)
  a_s = jnp.dot(h, as1_ref[...] * m, preferred_element_type=f32)
  a_d = jnp.dot(h, ad1_ref[...] * m, preferred_element_type=f32)
  row1_ref[:, 0:H1] = h
  row1_ref[:, H1:H1 + HEADS] = a_s
  row1_ref[:, H1 + HEADS:R1] = jnp.zeros((BLK, HEADS), f32)
  adt_ref[:, 0:HEADS] = a_d
  adt_ref[:, HEADS:16] = jnp.zeros((BLK, 8), f32)


def _phase_a(x, w1, as1c, ad1c):
  return pl.pallas_call(
      _phase_a_body,
      grid=(N // BLK,),
      in_specs=[
          pl.BlockSpec((BLK, NFEAT), lambda i: (i, 0)),
          pl.BlockSpec((NFEAT, H1), lambda i: (0, 0)),
          pl.BlockSpec((H1, 1), lambda i: (0, 0)),
          pl.BlockSpec((H1, 1), lambda i: (0, 0)),
      ],
      out_specs=[
          pl.BlockSpec((BLK, R1), lambda i: (i, 0)),
          pl.BlockSpec((BLK, 16), lambda i: (i, 0)),
      ],
      out_shape=[
          jax.ShapeDtypeStruct((N, R1), f32),
          jax.ShapeDtypeStruct((N, 16), f32),
      ],
  )(x, w1, as1c, ad1c)


def _leaky_exp(a):
  return jnp.exp(jnp.where(a >= 0, a, a * 0.2))


def _sc1_body(src_hbm, dst_hbm, row1_hbm, adt_hbm, out_hbm,
              rows_v, adr_v, sidx_v, didx_v, ebuf_v, acc, sem):
  cid = lax.axis_index("c")
  sid = lax.axis_index("s")
  wid = sid * NC + cid

  def _z(r, _):
    for j in range(R1 // LN):
      rows_v[r, pl.ds(j * LN, LN)] = jnp.zeros((LN,), f32)
    return 0
  lax.fori_loop(0, K, _z, 0)
  for z in range(RPT // K):
    pltpu.sync_copy(rows_v, acc.at[pl.ds(sid * RPT + z * K, K)])

  for r in [0] + list(range(HEADS + 1, LN)):
    ebuf_v[r, :] = jnp.zeros((LN,), f32)

  plsc.subcore_barrier()

  iota16 = lax.iota(i32, LN)
  ebase0 = wid * EPT

  def _batch(b, _):
    eb = ebase0 + b * K
    pltpu.sync_copy(src_hbm.at[pl.ds(eb, K)], sidx_v)
    pltpu.sync_copy(dst_hbm.at[pl.ds(eb, K)], didx_v)
    c1 = pltpu.async_copy(row1_hbm.at[sidx_v], rows_v, sem)
    c2 = pltpu.async_copy(adt_hbm.at[didx_v], adr_v, sem)
    c1.wait()
    c2.wait()

    def _group(g, _):
      eids = g * LN + iota16
      for k in range(HEADS):
        asr = plsc.load_gather(rows_v, [eids, jnp.full((LN,), H1 + k, i32)])
        ads = plsc.load_gather(adr_v, [eids, jnp.full((LN,), k, i32)])
        ebuf_v[k + 1, :] = _leaky_exp(asr + ads)
      for j in range(LN):
        jc = jnp.full((LN,), j, i32)
        ecol = plsc.load_gather(ebuf_v, [iota16, jc])
        r = g * LN + j
        rows_v[r, pl.ds(H1, LN)] = ecol
        for k in range(HEADS):
          bc = plsc.load_gather(ebuf_v, [jnp.full((LN,), k + 1, i32), jc])
          rows_v[r, pl.ds(k * LN, LN)] = rows_v[r, pl.ds(k * LN, LN)] * bc
      return 0
    lax.fori_loop(0, K // LN, _group, 0)

    pltpu.sync_copy(rows_v, acc.at[didx_v], add=True)
    return 0

  lax.fori_loop(0, NB, _batch, 0)

  plsc.subcore_barrier()
  pltpu.sync_copy(acc.at[pl.ds(sid * RPT, RPT)],
                  out_hbm.at[cid, pl.ds(sid * RPT, RPT)])


def _sc_layer1(src, dst, row1, adt):
  mesh = plsc.VectorSubcoreMesh(core_axis_name="c", subcore_axis_name="s",
                                num_cores=NC, num_subcores=NS)
  kern = functools.partial(
      pl.kernel,
      out_type=jax.ShapeDtypeStruct((NC, NPAD, R1), f32),
      mesh=mesh,
      compiler_params=pltpu.CompilerParams(use_tc_tiling_on_sc=False,
                                           needs_layout_passes=False),
      scratch_types=[
          pltpu.VMEM((K, R1), f32),
          pltpu.VMEM((K, 16), f32),
          pltpu.VMEM((K,), i32),
          pltpu.VMEM((K,), i32),
          pltpu.VMEM((LN, LN), f32),
          pltpu.VMEM_SHARED((NPAD, R1), f32),
          pltpu.SemaphoreType.DMA,
      ],
  )(_sc1_body)
  return kern(src, dst, row1, adt)


def _sc2_body(src_hbm, dst_hbm, row2_hbm, meta_hbm, out_hbm,
              rows_v, mr_v, sidx_v, didx_v, ebuf_v, acc, sem):
  cid = lax.axis_index("c")
  sid = lax.axis_index("s")
  wid = sid * NC + cid

  def _z(r, _):
    for j in range(R2 // LN):
      rows_v[r, pl.ds(j * LN, LN)] = jnp.zeros((LN,), f32)
    return 0
  lax.fori_loop(0, K, _z, 0)
  for z in range(RPT // K):
    pltpu.sync_copy(rows_v, acc.at[pl.ds(sid * RPT + z * K, K)])

  plsc.subcore_barrier()

  iota16 = lax.iota(i32, LN)
  lane_is8 = iota16 == 8
  ebase0 = wid * EPT

  def _batch(b, _):
    eb = ebase0 + b * K
    pltpu.sync_copy(src_hbm.at[pl.ds(eb, K)], sidx_v)
    pltpu.sync_copy(dst_hbm.at[pl.ds(eb, K)], didx_v)
    c1 = pltpu.async_copy(row2_hbm.at[sidx_v], rows_v, sem)
    c2 = pltpu.async_copy(meta_hbm.at[didx_v], mr_v, sem)
    c1.wait()
    c2.wait()

    def _group(g, _):
      eids = g * LN + iota16
      asr = plsc.load_gather(rows_v, [eids, jnp.full((LN,), NCLASS, i32)])
      ads = plsc.load_gather(mr_v, [eids, jnp.full((LN,), 1, i32)])
      ebuf_v[1, :] = _leaky_exp(asr + ads)
      for j in range(LN):
        bc = plsc.load_gather(ebuf_v, [jnp.full((LN,), 1, i32),
                                       jnp.full((LN,), j, i32)])
        r = g * LN + j
        rows_v[r, pl.ds(0, LN)] = rows_v[r, pl.ds(0, LN)] * bc
        rows_v[r, pl.ds(LN, LN)] = rows_v[r, pl.ds(LN, LN)] * bc
        v2 = rows_v[r, pl.ds(2 * LN, LN)] * bc
        rows_v[r, pl.ds(2 * LN, LN)] = jnp.where(lane_is8, bc, v2)
      return 0
    lax.fori_loop(0, K // LN, _group, 0)

    pltpu.sync_copy(rows_v, acc.at[didx_v], add=True)
    return 0

  lax.fori_loop(0, NB, _batch, 0)

  plsc.subcore_barrier()
  pltpu.sync_copy(acc.at[pl.ds(sid * RPT, RPT)],
                  out_hbm.at[cid, pl.ds(sid * RPT, RPT)])


def _sc_layer2(src, dst, row2, meta2):
  mesh = plsc.VectorSubcoreMesh(core_axis_name="c", subcore_axis_name="s",
                                num_cores=NC, num_subcores=NS)
  kern = functools.partial(
      pl.kernel,
      out_type=jax.ShapeDtypeStruct((NC, NPAD, R2), f32),
      mesh=mesh,
      compiler_params=pltpu.CompilerParams(use_tc_tiling_on_sc=False,
                                           needs_layout_passes=False),
      scratch_types=[
          pltpu.VMEM((K, R2), f32),
          pltpu.VMEM((K, 16), f32),
          pltpu.VMEM((K,), i32),
          pltpu.VMEM((K,), i32),
          pltpu.VMEM((2, LN), f32),
          pltpu.VMEM_SHARED((NPAD, R2), f32),
          pltpu.SemaphoreType.DMA,
      ],
  )(_sc2_body)
  return kern(src, dst, row2, meta2)


def _phase_c_body(p_ref, row1_ref, adt_ref, w2_ref, as2_ref, ad2_ref, b1_ref,
                  feat_ref, row2_ref, meta_ref):
  s = p_ref[0] + p_ref[1]
  h = row1_ref[:, 0:H1]
  a_s = row1_ref[:, H1:H1 + HEADS]
  a_d = adt_ref[:, 0:HEADS]
  e_self = _leaky_exp(a_s + a_d)
  bm = _head_bcast_mask(HEADS, H1, transpose=True)
  num = s[:, 0:H1] + h * jnp.dot(e_self, bm, preferred_element_type=f32)
  den = s[:, H1 + 1:H1 + 1 + HEADS] + e_self
  denb = jnp.dot(den, bm, preferred_element_type=f32)
  o = num / (denb + 1e-16) + b1_ref[...]
  feat = jnp.where(o > 0, o, jnp.exp(jnp.minimum(o, 0.0)) - 1.0)
  feat_ref[...] = feat
  g = jnp.dot(feat, w2_ref[...], preferred_element_type=f32)
  a_s2 = jnp.dot(g, as2_ref[...], preferred_element_type=f32)
  a_d2 = jnp.dot(g, ad2_ref[...], preferred_element_type=f32)
  row2_ref[:, 0:NCLASS] = g
  row2_ref[:, NCLASS:NCLASS + 1] = a_s2
  row2_ref[:, NCLASS + 1:R2] = jnp.zeros((BLK, R2 - NCLASS - 1), f32)
  meta_ref[:, 0:1] = a_s2
  meta_ref[:, 1:2] = a_d2
  meta_ref[:, 2:16] = jnp.zeros((BLK, 14), f32)


def _phase_c(p1, row1, adt, w2, as2c, ad2c, b1r):
  return pl.pallas_call(
      _phase_c_body,
      grid=(N // BLK,),
      in_specs=[
          pl.BlockSpec((NC, BLK, R1), lambda i: (0, i, 0)),
          pl.BlockSpec((BLK, R1), lambda i: (i, 0)),
          pl.BlockSpec((BLK, 16), lambda i: (i, 0)),
          pl.BlockSpec((H1, NCLASS), lambda i: (0, 0)),
          pl.BlockSpec((NCLASS, 1), lambda i: (0, 0)),
          pl.BlockSpec((NCLASS, 1), lambda i: (0, 0)),
          pl.BlockSpec((1, H1), lambda i: (0, 0)),
      ],
      out_specs=[
          pl.BlockSpec((BLK, H1), lambda i: (i, 0)),
          pl.BlockSpec((BLK, R2), lambda i: (i, 0)),
          pl.BlockSpec((BLK, 16), lambda i: (i, 0)),
      ],
      out_shape=[
          jax.ShapeDtypeStruct((N, H1), f32),
          jax.ShapeDtypeStruct((N, R2), f32),
          jax.ShapeDtypeStruct((N, 16), f32),
      ],
  )(p1, row1, adt, w2, as2c, ad2c, b1r)


def _phase_e_body(p_ref, row2_ref, meta_ref, b2_ref, out_ref):
  s = p_ref[0] + p_ref[1]
  g = row2_ref[:, 0:NCLASS]
  e_self = _leaky_exp(meta_ref[:, 0:1] + meta_ref[:, 1:2])
  num = s[:, 0:NCLASS] + g * e_self
  den = s[:, NCLASS:NCLASS + 1] + e_self
  o = num / (den + 1e-16) + b2_ref[...]
  m = jnp.max(o, axis=1, keepdims=True)
  lse = jnp.log(jnp.sum(jnp.exp(o - m), axis=1, keepdims=True)) + m
  out_ref[...] = o - lse


def _phase_e(p2, row2, meta2, b2r):
  return pl.pallas_call(
      _phase_e_body,
      grid=(N // BLK,),
      in_specs=[
          pl.BlockSpec((NC, BLK, R2), lambda i: (0, i, 0)),
          pl.BlockSpec((BLK, R2), lambda i: (i, 0)),
          pl.BlockSpec((BLK, 16), lambda i: (i, 0)),
          pl.BlockSpec((1, NCLASS), lambda i: (0, 0)),
      ],
      out_specs=pl.BlockSpec((BLK, NCLASS), lambda i: (i, 0)),
      out_shape=jax.ShapeDtypeStruct((N, NCLASS), f32),
  )(p2, row2, meta2, b2r)


@jax.jit
def kernel(x, edge_index, W1, att_src1, att_dst1, b1, W2, att_src2,
           att_dst2, b2):
  src = edge_index[0].astype(i32)
  dst = edge_index[1].astype(i32)
  as1c = att_src1.reshape(H1, 1)
  ad1c = att_dst1.reshape(H1, 1)
  row1, adt = _phase_a(x, W1, as1c, ad1c)
  p1 = _sc_layer1(src, dst, row1, adt)
  feat, row2, meta2 = _phase_c(p1, row1, adt, W2,
                               att_src2.reshape(NCLASS, 1),
                               att_dst2.reshape(NCLASS, 1),
                               b1.reshape(1, H1))
  p2 = _sc_layer2(src, dst, row2, meta2)
  out = _phase_e(p2, row2, meta2, b2.reshape(1, NCLASS))
  return (out, feat)
